# Initial kernel scaffold; baseline (speedup 1.0000x reference)
import jax
import jax.numpy as jnp
from jax.experimental import pallas as pl


def kernel(x):
    raise NotImplementedError("write your pallas kernel here")


# speedup vs baseline: 14.6053x; 14.6053x over previous
import jax
import jax.numpy as jnp
from jax.experimental import pallas as pl

_K = 256
_ROWS_PER_BLOCK = 8
_N = 32768


def _monotone_u32(x):
    u = jax.lax.bitcast_convert_type(x, jnp.uint32)
    return jnp.where(u < jnp.uint32(0x80000000),
                     u | jnp.uint32(0x80000000),
                     ~u)


def _topk_mask_block(x_ref, o_ref):
    x = x_ref[...]
    u = _monotone_u32(x)
    t = jnp.zeros((x.shape[0], 1), dtype=jnp.uint32)
    for bit in range(31, -1, -1):
        cand = t | jnp.uint32(1 << bit)
        cnt = jnp.sum((u >= cand).astype(jnp.int32), axis=1, keepdims=True)
        t = jnp.where(cnt >= _K, cand, t)
    o_ref[...] = jnp.where(u >= t, x, 0.0)


def kernel(x):
    m, n = x.shape
    grid = (m // _ROWS_PER_BLOCK,)
    return pl.pallas_call(
        _topk_mask_block,
        grid=grid,
        in_specs=[pl.BlockSpec((_ROWS_PER_BLOCK, n), lambda i: (i, 0))],
        out_specs=pl.BlockSpec((_ROWS_PER_BLOCK, n), lambda i: (i, 0)),
        out_shape=jax.ShapeDtypeStruct((m, n), x.dtype),
    )(x)
